# baseline (speedup 1.0000x reference)
import functools

import numpy as np
import jax
import jax.numpy as jnp
from jax import lax
from jax.experimental import pallas as pl
from jax.experimental.pallas import tpu as pltpu
from jax.experimental.pallas import tpu_sc as plsc

_F = 26
_D = 64
_B = 16384

_NC, _NS = 2, 16
_NW = _NC * _NS
_BPW = _B // _NW
_C = 32
_NCH = _BPW // _C
_RPC = _C * _F
_IPW = _BPW * _F
_SUB = (128, 128, 128, 128, 128, 128, 64)


def _col_offsets() -> np.ndarray:
    perm = np.arange(_F)
    np.random.RandomState(0).shuffle(perm)
    inv = np.empty(_F, dtype=np.int64)
    inv[perm] = np.arange(_F)
    return (inv * 100000).astype(np.int32)


_COLOFS = _col_offsets()


def _make_sc_kernel():
    mesh = plsc.VectorSubcoreMesh(core_axis_name="c", subcore_axis_name="s")

    @functools.partial(
        pl.kernel,
        mesh=mesh,
        out_type=jax.ShapeDtypeStruct((_B, _D), jnp.float32),
        scratch_types=[
            pltpu.VMEM((_IPW,), jnp.int32),
            pltpu.VMEM((_RPC, _D), jnp.float32),
            pltpu.VMEM((_C, _D), jnp.float32),
            pltpu.SemaphoreType.DMA,
        ],
        compiler_params=pltpu.CompilerParams(use_tc_tiling_on_sc=False),
    )
    def sc_kernel(idx_hbm, table_hbm, out_hbm, idx_v, rows_v, out_v, sem):
        cid = lax.axis_index("c")
        sid = lax.axis_index("s")
        wid = sid * _NC + cid
        ibase = wid * _IPW
        pltpu.sync_copy(idx_hbm.at[pl.ds(ibase, _IPW)], idx_v)

        for ch in range(_NCH):
            r0 = ch * _RPC
            copies = []
            off = 0
            for n in _SUB:
                copies.append(
                    pltpu.async_copy(
                        table_hbm.at[idx_v.at[pl.ds(r0 + off, n)]],
                        rows_v.at[pl.ds(off, n)],
                        sem,
                    )
                )
                off += n
            for cp in copies:
                cp.wait()

            def bag_body(i, carry):
                base = i * _F
                acc = [rows_v[base, pl.ds(d * 16, 16)] for d in range(4)]
                for f in range(1, _F):
                    r = base + f
                    for d in range(4):
                        acc[d] = acc[d] + rows_v[r, pl.ds(d * 16, 16)]
                for d in range(4):
                    out_v[i, pl.ds(d * 16, 16)] = acc[d]
                return carry

            lax.fori_loop(0, _C, bag_body, 0)
            pltpu.sync_copy(out_v, out_hbm.at[pl.ds(wid * _BPW + ch * _C, _C)])

    return sc_kernel


_sc_kernel = _make_sc_kernel()


@jax.jit
def kernel(x, embed_weight):
    idx = (x + jnp.asarray(_COLOFS)[None, :]).reshape(-1)
    return _sc_kernel(idx, embed_weight)
